# baseline (speedup 1.0000x reference)
import functools

import jax
import jax.numpy as jnp
from jax import lax
from jax.experimental import pallas as pl
from jax.experimental.pallas import tpu as pltpu
from jax.experimental.pallas import tpu_sc as plsc

_NC = 2
_NS = 16
_NW = _NC * _NS
_CH = 128


def _mesh():
    return plsc.VectorSubcoreMesh(core_axis_name="c", subcore_axis_name="s",
                                  num_cores=_NC, num_subcores=_NS)


def _fill(buf, value):
    rows, cols = buf.shape
    vec = jnp.full((16,), value, jnp.float32)

    def bi(i, carry):
        def bj(j, c2):
            buf[i, pl.ds(j * 16, 16)] = vec
            return c2
        return lax.fori_loop(0, cols // 16, bj, carry)

    lax.fori_loop(0, rows, bi, 0)


def _deg_call(dst2d, hd, acc_rows):
    rpt = dst2d.shape[0] // _NW
    zpt = acc_rows // _NS

    @functools.partial(
        pl.kernel,
        mesh=_mesh(),
        out_type=jax.ShapeDtypeStruct((_NC, acc_rows, hd), jnp.float32),
        scratch_types=[
            pltpu.VMEM((rpt, _CH), jnp.int32),
            pltpu.VMEM((_CH, hd), jnp.float32),
            pltpu.VMEM_SHARED((acc_rows, hd), jnp.float32),
        ],
        compiler_params=pltpu.CompilerParams(use_tc_tiling_on_sc=False),
    )
    def deg_kernel(dst_hbm, out_hbm, idx_v, val_v, acc):
        c = lax.axis_index("c")
        s = lax.axis_index("s")
        w = c * _NS + s
        _fill(val_v, 0.0)
        for k in range(zpt // _CH):
            pltpu.sync_copy(val_v, acc.at[pl.ds(s * zpt + k * _CH, _CH)])
        _fill(val_v, 1.0)
        pltpu.sync_copy(dst_hbm.at[pl.ds(w * rpt, rpt)], idx_v)
        plsc.subcore_barrier()

        def body(j, carry):
            pltpu.sync_copy(val_v, acc.at[idx_v.at[j]], add=True)
            return carry

        lax.fori_loop(0, rpt, body, 0)
        plsc.subcore_barrier()
        for k in range(zpt // _CH):
            pltpu.sync_copy(acc.at[pl.ds(s * zpt + k * _CH, _CH)],
                            out_hbm.at[c, pl.ds(s * zpt + k * _CH, _CH)])

    return deg_kernel(dst2d)


def _scatter_call(gs, src2d, dst2d, hd, acc_rows):
    rpt = src2d.shape[0] // _NS
    zpt = acc_rows // _NS
    nch = 4
    hr = rpt // nch

    @functools.partial(
        pl.kernel,
        mesh=_mesh(),
        out_type=jax.ShapeDtypeStruct((_NC, acc_rows, hd), jnp.float32),
        scratch_types=[
            pltpu.VMEM((hr, _CH), jnp.int32),
            pltpu.VMEM((hr, _CH), jnp.int32),
            pltpu.VMEM((_CH, hd), jnp.float32),
            pltpu.VMEM((_CH, hd), jnp.float32),
            pltpu.VMEM_SHARED((acc_rows, hd), jnp.float32),
            pltpu.VMEM_SHARED((acc_rows, hd), jnp.float32),
            pltpu.SemaphoreType.DMA,
            pltpu.SemaphoreType.DMA,
            pltpu.SemaphoreType.DMA,
            pltpu.SemaphoreType.DMA,
        ],
        compiler_params=pltpu.CompilerParams(use_tc_tiling_on_sc=False),
    )
    def scatter_kernel(gs_hbm, src_hbm, dst_hbm, out_hbm,
                       src_v, dst_v, buf0, buf1, gsm, acc, sem0, sem1, sem2, sem3):
        c = lax.axis_index("c")
        s = lax.axis_index("s")
        pltpu.sync_copy(gs_hbm.at[c, pl.ds(s * zpt, zpt)],
                        gsm.at[pl.ds(s * zpt, zpt)])
        _fill(buf0, 0.0)
        for k in range(zpt // _CH):
            pltpu.sync_copy(buf0, acc.at[pl.ds(s * zpt + k * _CH, _CH)])
        plsc.subcore_barrier()

        for half in range(nch):
            base = s * rpt + half * hr
            pltpu.sync_copy(src_hbm.at[pl.ds(base, hr)], src_v)
            pltpu.sync_copy(dst_hbm.at[pl.ds(base, hr)], dst_v)
            pltpu.async_copy(gsm.at[src_v.at[0]], buf0, sem0)
            pltpu.async_copy(gsm.at[src_v.at[1]], buf1, sem1)

            def body(k, carry):
                j = 2 * k
                pltpu.make_async_copy(gsm.at[src_v.at[j]], buf0, sem0).wait()
                d0 = pltpu.async_copy(buf0, acc.at[dst_v.at[j]], sem2, add=True)
                pltpu.make_async_copy(gsm.at[src_v.at[j + 1]], buf1, sem1).wait()
                d1 = pltpu.async_copy(buf1, acc.at[dst_v.at[j + 1]], sem3, add=True)
                d0.wait()
                pltpu.async_copy(gsm.at[src_v.at[j + 2]], buf0, sem0)
                d1.wait()
                pltpu.async_copy(gsm.at[src_v.at[j + 3]], buf1, sem1)
                return carry

            lax.fori_loop(0, hr // 2 - 1, body, 0)
            j = hr - 2
            pltpu.make_async_copy(gsm.at[src_v.at[j]], buf0, sem0).wait()
            d0 = pltpu.async_copy(buf0, acc.at[dst_v.at[j]], sem2, add=True)
            pltpu.make_async_copy(gsm.at[src_v.at[j + 1]], buf1, sem1).wait()
            d1 = pltpu.async_copy(buf1, acc.at[dst_v.at[j + 1]], sem3, add=True)
            d0.wait()
            d1.wait()
        plsc.subcore_barrier()
        for k in range(zpt // _CH):
            pltpu.sync_copy(acc.at[pl.ds(s * zpt + k * _CH, _CH)],
                            out_hbm.at[c, pl.ds(s * zpt + k * _CH, _CH)])

    return scatter_kernel(gs, src2d, dst2d)


def _matmul2_call(x, wc, wl, blk):
    n, d = x.shape
    grid = (n // blk,)

    def body(x_ref, wc_ref, wl_ref, g_ref, l_ref):
        h = x_ref[...]
        g_ref[...] = jnp.dot(h, wc_ref[...], preferred_element_type=jnp.float32)
        l_ref[...] = jnp.dot(h, wl_ref[...], preferred_element_type=jnp.float32)

    return pl.pallas_call(
        body,
        grid=grid,
        in_specs=[
            pl.BlockSpec((blk, d), lambda i: (i, 0)),
            pl.BlockSpec((d, d), lambda i: (0, 0)),
            pl.BlockSpec((d, d), lambda i: (0, 0)),
        ],
        out_specs=[
            pl.BlockSpec((blk, d), lambda i: (i, 0)),
            pl.BlockSpec((blk, d), lambda i: (i, 0)),
        ],
        out_shape=[
            jax.ShapeDtypeStruct((n, d), jnp.float32),
            jax.ShapeDtypeStruct((n, d), jnp.float32),
        ],
    )(x, wc, wl)


def _layer0_call(g, l, degp, dw, b, blk, acc_rows):
    n, d = g.shape
    hd = d // 2
    grid = (n // blk,)

    def body(g_ref, l_ref, degp_ref, b_ref, gs_ref, z_ref, dinv_ref):
        dsum = degp_ref[0, :, 0:1] + degp_ref[1, :, 0:1]
        dv = lax.rsqrt(dsum + 1.0)
        g = g_ref[...]
        gsall = dv * g
        gs_ref[0, :, :] = gsall[:, :hd]
        gs_ref[1, :, :] = gsall[:, hd:]
        z_ref[...] = l_ref[...] + b_ref[...] + (dv * dv) * g
        dinv_ref[...] = dv

    return pl.pallas_call(
        body,
        grid=grid,
        in_specs=[
            pl.BlockSpec((blk, d), lambda i: (i, 0)),
            pl.BlockSpec((blk, d), lambda i: (i, 0)),
            pl.BlockSpec((_NC, blk, dw), lambda i: (0, i, 0)),
            pl.BlockSpec((1, d), lambda i: (0, 0)),
        ],
        out_specs=[
            pl.BlockSpec((_NC, blk, hd), lambda i: (0, i, 0)),
            pl.BlockSpec((blk, d), lambda i: (i, 0)),
            pl.BlockSpec((blk, 1), lambda i: (i, 0)),
        ],
        out_shape=[
            jax.ShapeDtypeStruct((_NC, acc_rows, hd), jnp.float32),
            jax.ShapeDtypeStruct((n, d), jnp.float32),
            jax.ShapeDtypeStruct((n, 1), jnp.float32),
        ],
    )(g, l, degp, b)


def _layer_call(p, z, dinv, wc, wl, b, blk, acc_rows):
    n, d = z.shape
    hd = d // 2
    grid = (n // blk,)

    def body(p_ref, zin_ref, dinv_ref, wc_ref, wl_ref, b_ref, gs_ref, zout_ref):
        dv = dinv_ref[...]
        psum = jnp.concatenate([p_ref[0], p_ref[1]], axis=1)
        h = jnp.maximum(dv * psum + zin_ref[...], 0.0)
        g = jnp.dot(h, wc_ref[...], preferred_element_type=jnp.float32)
        gsall = dv * g
        gs_ref[0, :, :] = gsall[:, :hd]
        gs_ref[1, :, :] = gsall[:, hd:]
        zout_ref[...] = (jnp.dot(h, wl_ref[...], preferred_element_type=jnp.float32)
                         + b_ref[...] + (dv * dv) * g)

    return pl.pallas_call(
        body,
        grid=grid,
        in_specs=[
            pl.BlockSpec((_NC, blk, hd), lambda i: (0, i, 0)),
            pl.BlockSpec((blk, d), lambda i: (i, 0)),
            pl.BlockSpec((blk, 1), lambda i: (i, 0)),
            pl.BlockSpec((d, d), lambda i: (0, 0)),
            pl.BlockSpec((d, d), lambda i: (0, 0)),
            pl.BlockSpec((1, d), lambda i: (0, 0)),
        ],
        out_specs=[
            pl.BlockSpec((_NC, blk, hd), lambda i: (0, i, 0)),
            pl.BlockSpec((blk, d), lambda i: (i, 0)),
        ],
        out_shape=[
            jax.ShapeDtypeStruct((_NC, acc_rows, hd), jnp.float32),
            jax.ShapeDtypeStruct((n, d), jnp.float32),
        ],
    )(p, z, dinv, wc, wl, b)


def _final_call(p, z, dinv, blk):
    n, d = z.shape
    hd = d // 2
    grid = (n // blk,)

    def body(p_ref, zin_ref, dinv_ref, out_ref):
        psum = jnp.concatenate([p_ref[0], p_ref[1]], axis=1)
        out_ref[...] = dinv_ref[...] * psum + zin_ref[...]

    return pl.pallas_call(
        body,
        grid=grid,
        in_specs=[
            pl.BlockSpec((_NC, blk, hd), lambda i: (0, i, 0)),
            pl.BlockSpec((blk, d), lambda i: (i, 0)),
            pl.BlockSpec((blk, 1), lambda i: (i, 0)),
        ],
        out_specs=pl.BlockSpec((blk, d), lambda i: (i, 0)),
        out_shape=jax.ShapeDtypeStruct((n, d), jnp.float32),
    )(p, z, dinv)


def kernel(x, edge_index, Wc, bc, Wl, bl):
    n, d = x.shape
    e = edge_index.shape[1]
    nl = Wc.shape[0]

    ept = -(-e // (_NW * _CH * 8)) * (_CH * 8)
    epad = ept * _NW
    src = jnp.concatenate(
        [edge_index[0], jnp.zeros((epad - e,), jnp.int32)]).reshape(-1, _CH)
    dst = jnp.concatenate(
        [edge_index[1], jnp.full((epad - e,), n, jnp.int32)]).reshape(-1, _CH)

    acc_rows = -(-(n + 1) // (_NS * _CH)) * (_NS * _CH)
    blk = 1000
    hd = d // 2
    dw = 32

    degp = _deg_call(dst, dw, acc_rows)
    g0, l0 = _matmul2_call(x, Wc[0], Wl[0], blk)
    b0 = (bc[0] + bl[0]).reshape(1, d)
    gs, z, dinv = _layer0_call(g0, l0, degp, dw, b0, blk, acc_rows)
    for layer in range(1, nl):
        p = _scatter_call(gs, src, dst, hd, acc_rows)
        bsum = (bc[layer] + bl[layer]).reshape(1, d)
        gs, z = _layer_call(p, z, dinv, Wc[layer], Wl[layer], bsum, blk, acc_rows)
    p = _scatter_call(gs, src, dst, hd, acc_rows)
    return _final_call(p, z, dinv, blk)


# speedup vs baseline: 1.1027x; 1.1027x over previous
import functools

import jax
import jax.numpy as jnp
from jax import lax
from jax.experimental import pallas as pl
from jax.experimental.pallas import tpu as pltpu
from jax.experimental.pallas import tpu_sc as plsc

_NC = 2
_NS = 16
_NW = _NC * _NS
_CH = 128


def _mesh():
    return plsc.VectorSubcoreMesh(core_axis_name="c", subcore_axis_name="s",
                                  num_cores=_NC, num_subcores=_NS)


def _fill(buf, value):
    rows, cols = buf.shape
    vec = jnp.full((16,), value, jnp.float32)

    def bi(i, carry):
        def bj(j, c2):
            buf[i, pl.ds(j * 16, 16)] = vec
            return c2
        return lax.fori_loop(0, cols // 16, bj, carry)

    lax.fori_loop(0, rows, bi, 0)


def _deg_call(dst2d, hd, acc_rows):
    rpt = dst2d.shape[0] // _NW
    zpt = acc_rows // _NS

    @functools.partial(
        pl.kernel,
        mesh=_mesh(),
        out_type=jax.ShapeDtypeStruct((_NC, acc_rows, hd), jnp.float32),
        scratch_types=[
            pltpu.VMEM((rpt, _CH), jnp.int32),
            pltpu.VMEM((_CH, hd), jnp.float32),
            pltpu.VMEM_SHARED((acc_rows, hd), jnp.float32),
        ],
        compiler_params=pltpu.CompilerParams(use_tc_tiling_on_sc=False),
    )
    def deg_kernel(dst_hbm, out_hbm, idx_v, val_v, acc):
        c = lax.axis_index("c")
        s = lax.axis_index("s")
        w = c * _NS + s
        _fill(val_v, 0.0)
        for k in range(zpt // _CH):
            pltpu.sync_copy(val_v, acc.at[pl.ds(s * zpt + k * _CH, _CH)])
        _fill(val_v, 1.0)
        pltpu.sync_copy(dst_hbm.at[pl.ds(w * rpt, rpt)], idx_v)
        plsc.subcore_barrier()

        def body(j, carry):
            pltpu.sync_copy(val_v, acc.at[idx_v.at[j]], add=True)
            return carry

        lax.fori_loop(0, rpt, body, 0)
        plsc.subcore_barrier()
        for k in range(zpt // _CH):
            pltpu.sync_copy(acc.at[pl.ds(s * zpt + k * _CH, _CH)],
                            out_hbm.at[c, pl.ds(s * zpt + k * _CH, _CH)])

    return deg_kernel(dst2d)


def _scatter_call(gs, src2d, dst2d, hd, acc_rows):
    rpt = src2d.shape[0] // _NS
    zpt = acc_rows // _NS
    nch = 2
    hr = rpt // nch

    @functools.partial(
        pl.kernel,
        mesh=_mesh(),
        out_type=jax.ShapeDtypeStruct((_NC, acc_rows, hd), jnp.float32),
        scratch_types=[
            pltpu.VMEM((hr, _CH), jnp.int32),
            pltpu.VMEM((hr, _CH), jnp.int32),
            pltpu.VMEM((_CH, hd), jnp.float32),
            pltpu.VMEM((_CH, hd), jnp.float32),
            pltpu.VMEM_SHARED((acc_rows, hd), jnp.float32),
            pltpu.VMEM_SHARED((acc_rows, hd), jnp.float32),
            pltpu.SemaphoreType.DMA,
            pltpu.SemaphoreType.DMA,
            pltpu.SemaphoreType.DMA,
            pltpu.SemaphoreType.DMA,
        ],
        compiler_params=pltpu.CompilerParams(use_tc_tiling_on_sc=False),
    )
    def scatter_kernel(gs_hbm, src_hbm, dst_hbm, out_hbm,
                       src_v, dst_v, buf0, buf1, gsm, acc, sem0, sem1, sem2, sem3):
        c = lax.axis_index("c")
        s = lax.axis_index("s")
        pltpu.sync_copy(gs_hbm.at[c, pl.ds(s * zpt, zpt)],
                        gsm.at[pl.ds(s * zpt, zpt)])
        _fill(buf0, 0.0)
        for k in range(zpt // _CH):
            pltpu.sync_copy(buf0, acc.at[pl.ds(s * zpt + k * _CH, _CH)])
        plsc.subcore_barrier()

        for half in range(nch):
            base = s * rpt + half * hr
            pltpu.sync_copy(src_hbm.at[pl.ds(base, hr)], src_v)
            pltpu.sync_copy(dst_hbm.at[pl.ds(base, hr)], dst_v)
            pltpu.async_copy(gsm.at[src_v.at[0]], buf0, sem0)
            pltpu.async_copy(gsm.at[src_v.at[1]], buf1, sem1)

            def body(k, carry):
                j = 2 * k
                pltpu.make_async_copy(gsm.at[src_v.at[j]], buf0, sem0).wait()
                pltpu.sync_copy(buf0, acc.at[dst_v.at[j]], add=True)
                pltpu.async_copy(gsm.at[src_v.at[j + 2]], buf0, sem0)
                pltpu.make_async_copy(gsm.at[src_v.at[j + 1]], buf1, sem1).wait()
                pltpu.sync_copy(buf1, acc.at[dst_v.at[j + 1]], add=True)
                pltpu.async_copy(gsm.at[src_v.at[j + 3]], buf1, sem1)
                return carry

            lax.fori_loop(0, hr // 2 - 1, body, 0)
            j = hr - 2
            pltpu.make_async_copy(gsm.at[src_v.at[j]], buf0, sem0).wait()
            pltpu.sync_copy(buf0, acc.at[dst_v.at[j]], add=True)
            pltpu.make_async_copy(gsm.at[src_v.at[j + 1]], buf1, sem1).wait()
            pltpu.sync_copy(buf1, acc.at[dst_v.at[j + 1]], add=True)
        plsc.subcore_barrier()
        for k in range(zpt // _CH):
            pltpu.sync_copy(acc.at[pl.ds(s * zpt + k * _CH, _CH)],
                            out_hbm.at[c, pl.ds(s * zpt + k * _CH, _CH)])

    return scatter_kernel(gs, src2d, dst2d)


def _matmul2_call(x, wc, wl, blk):
    n, d = x.shape
    grid = (n // blk,)

    def body(x_ref, wc_ref, wl_ref, g_ref, l_ref):
        h = x_ref[...]
        g_ref[...] = jnp.dot(h, wc_ref[...], preferred_element_type=jnp.float32)
        l_ref[...] = jnp.dot(h, wl_ref[...], preferred_element_type=jnp.float32)

    return pl.pallas_call(
        body,
        grid=grid,
        in_specs=[
            pl.BlockSpec((blk, d), lambda i: (i, 0)),
            pl.BlockSpec((d, d), lambda i: (0, 0)),
            pl.BlockSpec((d, d), lambda i: (0, 0)),
        ],
        out_specs=[
            pl.BlockSpec((blk, d), lambda i: (i, 0)),
            pl.BlockSpec((blk, d), lambda i: (i, 0)),
        ],
        out_shape=[
            jax.ShapeDtypeStruct((n, d), jnp.float32),
            jax.ShapeDtypeStruct((n, d), jnp.float32),
        ],
    )(x, wc, wl)


def _layer0_call(g, l, degp, dw, b, blk, acc_rows):
    n, d = g.shape
    hd = d // 2
    grid = (n // blk,)

    def body(g_ref, l_ref, degp_ref, b_ref, gs_ref, z_ref, dinv_ref):
        dsum = degp_ref[0, :, 0:1] + degp_ref[1, :, 0:1]
        dv = lax.rsqrt(dsum + 1.0)
        g = g_ref[...]
        gsall = dv * g
        gs_ref[0, :, :] = gsall[:, :hd]
        gs_ref[1, :, :] = gsall[:, hd:]
        z_ref[...] = l_ref[...] + b_ref[...] + (dv * dv) * g
        dinv_ref[...] = dv

    return pl.pallas_call(
        body,
        grid=grid,
        in_specs=[
            pl.BlockSpec((blk, d), lambda i: (i, 0)),
            pl.BlockSpec((blk, d), lambda i: (i, 0)),
            pl.BlockSpec((_NC, blk, dw), lambda i: (0, i, 0)),
            pl.BlockSpec((1, d), lambda i: (0, 0)),
        ],
        out_specs=[
            pl.BlockSpec((_NC, blk, hd), lambda i: (0, i, 0)),
            pl.BlockSpec((blk, d), lambda i: (i, 0)),
            pl.BlockSpec((blk, 1), lambda i: (i, 0)),
        ],
        out_shape=[
            jax.ShapeDtypeStruct((_NC, acc_rows, hd), jnp.float32),
            jax.ShapeDtypeStruct((n, d), jnp.float32),
            jax.ShapeDtypeStruct((n, 1), jnp.float32),
        ],
    )(g, l, degp, b)


def _layer_call(p, z, dinv, wc, wl, b, blk, acc_rows):
    n, d = z.shape
    hd = d // 2
    grid = (n // blk,)

    def body(p_ref, zin_ref, dinv_ref, wc_ref, wl_ref, b_ref, gs_ref, zout_ref):
        dv = dinv_ref[...]
        psum = jnp.concatenate([p_ref[0], p_ref[1]], axis=1)
        h = jnp.maximum(dv * psum + zin_ref[...], 0.0)
        g = jnp.dot(h, wc_ref[...], preferred_element_type=jnp.float32)
        gsall = dv * g
        gs_ref[0, :, :] = gsall[:, :hd]
        gs_ref[1, :, :] = gsall[:, hd:]
        zout_ref[...] = (jnp.dot(h, wl_ref[...], preferred_element_type=jnp.float32)
                         + b_ref[...] + (dv * dv) * g)

    return pl.pallas_call(
        body,
        grid=grid,
        in_specs=[
            pl.BlockSpec((_NC, blk, hd), lambda i: (0, i, 0)),
            pl.BlockSpec((blk, d), lambda i: (i, 0)),
            pl.BlockSpec((blk, 1), lambda i: (i, 0)),
            pl.BlockSpec((d, d), lambda i: (0, 0)),
            pl.BlockSpec((d, d), lambda i: (0, 0)),
            pl.BlockSpec((1, d), lambda i: (0, 0)),
        ],
        out_specs=[
            pl.BlockSpec((_NC, blk, hd), lambda i: (0, i, 0)),
            pl.BlockSpec((blk, d), lambda i: (i, 0)),
        ],
        out_shape=[
            jax.ShapeDtypeStruct((_NC, acc_rows, hd), jnp.float32),
            jax.ShapeDtypeStruct((n, d), jnp.float32),
        ],
    )(p, z, dinv, wc, wl, b)


def _final_call(p, z, dinv, blk):
    n, d = z.shape
    hd = d // 2
    grid = (n // blk,)

    def body(p_ref, zin_ref, dinv_ref, out_ref):
        psum = jnp.concatenate([p_ref[0], p_ref[1]], axis=1)
        out_ref[...] = dinv_ref[...] * psum + zin_ref[...]

    return pl.pallas_call(
        body,
        grid=grid,
        in_specs=[
            pl.BlockSpec((_NC, blk, hd), lambda i: (0, i, 0)),
            pl.BlockSpec((blk, d), lambda i: (i, 0)),
            pl.BlockSpec((blk, 1), lambda i: (i, 0)),
        ],
        out_specs=pl.BlockSpec((blk, d), lambda i: (i, 0)),
        out_shape=jax.ShapeDtypeStruct((n, d), jnp.float32),
    )(p, z, dinv)


def kernel(x, edge_index, Wc, bc, Wl, bl):
    n, d = x.shape
    e = edge_index.shape[1]
    nl = Wc.shape[0]

    ept = -(-e // (_NW * _CH * 8)) * (_CH * 8)
    epad = ept * _NW
    src = jnp.concatenate(
        [edge_index[0], jnp.zeros((epad - e,), jnp.int32)]).reshape(-1, _CH)
    dst = jnp.concatenate(
        [edge_index[1], jnp.full((epad - e,), n, jnp.int32)]).reshape(-1, _CH)

    acc_rows = -(-(n + 1) // (_NS * _CH)) * (_NS * _CH)
    blk = 1000
    hd = d // 2
    dw = 32

    degp = _deg_call(dst, dw, acc_rows)
    g0, l0 = _matmul2_call(x, Wc[0], Wl[0], blk)
    b0 = (bc[0] + bl[0]).reshape(1, d)
    gs, z, dinv = _layer0_call(g0, l0, degp, dw, b0, blk, acc_rows)
    for layer in range(1, nl):
        p = _scatter_call(gs, src, dst, hd, acc_rows)
        bsum = (bc[layer] + bl[layer]).reshape(1, d)
        gs, z = _layer_call(p, z, dinv, Wc[layer], Wl[layer], bsum, blk, acc_rows)
    p = _scatter_call(gs, src, dst, hd, acc_rows)
    return _final_call(p, z, dinv, blk)
